# baseline (speedup 1.0000x reference)
import jax
import jax.numpy as jnp
from jax import lax
from jax.experimental import pallas as pl
from jax.experimental.pallas import tpu as pltpu
from jax.experimental.pallas import tpu_sc as plsc

_BATCH = 16384
_FEAT = 64
_CLS = 100000
_CLS_ALIGNED = 99968
_NC, _NS, _L = 2, 16, 16
_NW = _NC * _NS
_RPW = _FEAT // _NW
_LCH = 2048
_NCH = _BATCH // _LCH


def _center_loss_tec(feat_hbm, lab_hbm, cent_hbm, out_hbm,
                     idx16_v, lab_v, f_v, c_v, tail_v, acc_v,
                     csem, c2sem, fsem, lsem):
    wid = lax.axis_index("s") * _NC + lax.axis_index("c")
    j0 = wid * _RPW
    lanes = lax.iota(jnp.int32, _L)
    idx16_v[...] = jnp.full((_L,), j0, jnp.int32) + lax.shift_right_logical(lanes, 3)

    blk = (j0 // 8) * 8
    zr = jnp.zeros((_L,), jnp.int32)
    ca_v = jnp.full((_L,), _CLS_ALIGNED, jnp.int32)
    z = jnp.zeros((_L,), jnp.float32)
    acc = z

    for unit in range(_RPW):
        row = idx16_v.at[pl.ds(unit * 8, 1)]
        cmain = pltpu.async_copy(
            cent_hbm.at[row, pl.ds(0, 49920)],
            c_v.at[:, pl.ds(0, 49920)], csem)
        cmain2 = pltpu.async_copy(
            cent_hbm.at[row, pl.ds(49920, _CLS_ALIGNED - 49920)],
            c_v.at[:, pl.ds(49920, _CLS_ALIGNED - 49920)], c2sem)
        if unit == 0:
            ctail = pltpu.async_copy(
                cent_hbm.at[pl.ds(blk, 8), pl.ds(_CLS_ALIGNED,
                                                 _CLS - _CLS_ALIGNED)],
                tail_v, csem)
        fcp = pltpu.async_copy(feat_hbm.at[row], f_v, fsem)
        lcp0 = pltpu.async_copy(lab_hbm.at[pl.ds(0, _LCH)],
                                lab_v.at[0], lsem)
        fcp.wait()
        cmain.wait()
        cmain2.wait()
        if unit == 0:
            ctail.wait()
        r = j0 - blk + unit
        c_v[0, pl.ds(_CLS_ALIGNED, _L)] = tail_v[r, pl.ds(0, _L)]
        c_v[0, pl.ds(_CLS_ALIGNED + _L, _L)] = tail_v[r, pl.ds(_L, _L)]

        for k in range(_NCH):
            if k == 0:
                lcp0.wait()
            if k + 1 < _NCH:
                lnext = pltpu.async_copy(
                    lab_hbm.at[pl.ds((k + 1) * _LCH, _LCH)],
                    lab_v.at[(k + 1) % 2], lsem)

            def body(t, acc, k=k):
                lab = lab_v[k % 2, pl.ds(t * _L, _L)]
                g = plsc.load_gather(c_v, [zr, lab])
                f = f_v[0, pl.ds(k * _LCH + t * _L, _L)]
                d = f - g
                return acc + d * d

            acc = lax.fori_loop(0, _LCH // _L, body, acc, unroll=16)
            if k + 1 < _NCH:
                lnext.wait()

    acc_v[...] = acc
    pltpu.sync_copy(acc_v, out_hbm.at[wid])


def kernel(features, labels, centers):
    if labels.ndim > 1:
        labels = jnp.squeeze(labels, axis=-1)
    mesh = plsc.VectorSubcoreMesh(core_axis_name="c", subcore_axis_name="s")
    partials = pl.kernel(
        _center_loss_tec,
        out_type=jax.ShapeDtypeStruct((_NW, _L), jnp.float32),
        mesh=mesh,
        compiler_params=pltpu.CompilerParams(needs_layout_passes=False),
        scratch_types=[
            pltpu.VMEM((_L,), jnp.int32),
            pltpu.VMEM((2, _LCH), jnp.int32),
            pltpu.VMEM((1, _BATCH), jnp.float32),
            pltpu.VMEM((1, _CLS), jnp.float32),
            pltpu.VMEM((8, _CLS - _CLS_ALIGNED), jnp.float32),
            pltpu.VMEM((_L,), jnp.float32),
            pltpu.SemaphoreType.DMA,
            pltpu.SemaphoreType.DMA,
            pltpu.SemaphoreType.DMA,
            pltpu.SemaphoreType.DMA,
        ],
    )(features.T, labels.astype(jnp.int32), centers.T)
    return (jnp.sum(partials) / (2.0 * _BATCH)).astype(jnp.float32)


# speedup vs baseline: 1.1807x; 1.1807x over previous
import jax
import jax.numpy as jnp
from jax import lax
from jax.experimental import pallas as pl
from jax.experimental.pallas import tpu as pltpu
from jax.experimental.pallas import tpu_sc as plsc

_BATCH = 16384
_FEAT = 64
_CLS = 100000
_CLS_ALIGNED = 99968
_NC, _NS, _L = 2, 16, 16
_NW = _NC * _NS
_RPW = _FEAT // _NW
_LCH = 4096
_NCH = _BATCH // _LCH


def _center_loss_tec(feat_hbm, lab_hbm, cent_hbm, out_hbm,
                     idx16_v, lab_v, f_v, c_v, tail_v, acc_v,
                     csem, fsem, lsem):
    wid = lax.axis_index("s") * _NC + lax.axis_index("c")
    j0 = wid * _RPW
    lanes = lax.iota(jnp.int32, _L)
    idx16_v[...] = jnp.full((_L,), j0, jnp.int32) + lax.shift_right_logical(lanes, 3)

    blk = (j0 // 8) * 8
    zr = jnp.zeros((_L,), jnp.int32)
    ca_v = jnp.full((_L,), _CLS_ALIGNED, jnp.int32)
    z = jnp.zeros((_L,), jnp.float32)
    acc = z

    for unit in range(_RPW):
        row = idx16_v.at[pl.ds(unit * 8, 1)]
        cmain = pltpu.async_copy(
            cent_hbm.at[row, pl.ds(0, _CLS_ALIGNED)],
            c_v.at[:, pl.ds(0, _CLS_ALIGNED)], csem)
        if unit == 0:
            ctail = pltpu.async_copy(
                cent_hbm.at[pl.ds(blk, 8), pl.ds(_CLS_ALIGNED,
                                                 _CLS - _CLS_ALIGNED)],
                tail_v, csem)
        fcp = pltpu.async_copy(feat_hbm.at[row], f_v, fsem)
        lcp0 = pltpu.async_copy(lab_hbm.at[pl.ds(0, _LCH)],
                                lab_v.at[0], lsem)
        fcp.wait()
        cmain.wait()
        if unit == 0:
            ctail.wait()
        r = j0 - blk + unit
        c_v[0, pl.ds(_CLS_ALIGNED, _L)] = tail_v[r, pl.ds(0, _L)]
        c_v[0, pl.ds(_CLS_ALIGNED + _L, _L)] = tail_v[r, pl.ds(_L, _L)]

        for k in range(_NCH):
            if k == 0:
                lcp0.wait()
            if k + 1 < _NCH:
                lnext = pltpu.async_copy(
                    lab_hbm.at[pl.ds((k + 1) * _LCH, _LCH)],
                    lab_v.at[(k + 1) % 2], lsem)

            def body(t, acc, k=k):
                lab = lab_v[k % 2, pl.ds(t * _L, _L)]
                g = plsc.load_gather(c_v, [zr, lab])
                f = f_v[0, pl.ds(k * _LCH + t * _L, _L)]
                d = f - g
                return acc + d * d

            acc = lax.fori_loop(0, _LCH // _L, body, acc, unroll=8)
            if k + 1 < _NCH:
                lnext.wait()

    acc_v[...] = acc
    pltpu.sync_copy(acc_v, out_hbm.at[wid])


def kernel(features, labels, centers):
    if labels.ndim > 1:
        labels = jnp.squeeze(labels, axis=-1)
    mesh = plsc.VectorSubcoreMesh(core_axis_name="c", subcore_axis_name="s")
    partials = pl.kernel(
        _center_loss_tec,
        out_type=jax.ShapeDtypeStruct((_NW, _L), jnp.float32),
        mesh=mesh,
        compiler_params=pltpu.CompilerParams(needs_layout_passes=False),
        scratch_types=[
            pltpu.VMEM((_L,), jnp.int32),
            pltpu.VMEM((2, _LCH), jnp.int32),
            pltpu.VMEM((1, _BATCH), jnp.float32),
            pltpu.VMEM((1, _CLS), jnp.float32),
            pltpu.VMEM((8, _CLS - _CLS_ALIGNED), jnp.float32),
            pltpu.VMEM((_L,), jnp.float32),
            pltpu.SemaphoreType.DMA,
            pltpu.SemaphoreType.DMA,
            pltpu.SemaphoreType.DMA,
        ],
    )(features.T, labels.astype(jnp.int32), centers.T)
    return (jnp.sum(partials) / (2.0 * _BATCH)).astype(jnp.float32)
